# baseline (speedup 1.0000x reference)
import functools

import jax
import jax.numpy as jnp
from jax import lax
from jax.experimental import pallas as pl
from jax.experimental.pallas import tpu as pltpu
from jax.experimental.pallas import tpu_sc as plsc

CTX = 200
DIM = 64
B = 1024
L = 200
ROWS = B * L
NC, NS = 2, 16
NW = NC * NS
RPW = ROWS // NW
CHUNK = 800
NCHUNK = RPW // CHUNK
GB = 100
NGB = CHUNK // GB
LANES = 16
DREG = DIM // LANES

_mesh = plsc.VectorSubcoreMesh(core_axis_name="c", subcore_axis_name="s")


@functools.partial(
    pl.kernel,
    mesh=_mesh,
    out_type=jax.ShapeDtypeStruct((ROWS, DIM), jnp.float32),
    scratch_types=[
        pltpu.VMEM((CTX, DIM), jnp.float32),
        pltpu.VMEM((NGB, GB), jnp.int32),
        pltpu.VMEM((CHUNK, DIM), jnp.float32),
        pltpu.SemaphoreType.DMA,
    ],
    compiler_params=pltpu.CompilerParams(use_tc_tiling_on_sc=False),
)
def _emb(ids_hbm, tok_hbm, pos_hbm, out_hbm, pos_v, idx_v, g_v, sem):
    wid = lax.axis_index("s") * NC + lax.axis_index("c")
    pltpu.sync_copy(pos_hbm, pos_v)

    def chunk_body(c, carry):
        row0 = pl.multiple_of(wid * RPW + c * CHUNK, CHUNK)
        idx_row0 = pl.multiple_of(row0 // GB, NGB)
        pltpu.sync_copy(ids_hbm.at[pl.ds(idx_row0, NGB)], idx_v)
        copies = [
            pltpu.async_copy(
                tok_hbm.at[idx_v.at[j]], g_v.at[pl.ds(j * GB, GB)], sem
            )
            for j in range(NGB)
        ]
        for cp in copies:
            cp.wait()

        def row_body(r, rcarry):
            for d in range(DREG):
                s = pl.ds(d * LANES, LANES)
                p = pos_v[r, s]
                for q in range(CHUNK // CTX):
                    i = q * CTX + r
                    g_v[i, s] = g_v[i, s] + p
            return rcarry

        lax.fori_loop(0, CTX, row_body, 0)
        pltpu.sync_copy(g_v, out_hbm.at[pl.ds(row0, CHUNK)])
        return carry

    lax.fori_loop(0, NCHUNK, chunk_body, 0)


def kernel(input_ids, tok_table, pos_table):
    ids = input_ids.astype(jnp.int32).reshape(ROWS // GB, GB)
    out = _emb(ids, tok_table, pos_table)
    return out.reshape(B, L, DIM)
